# baseline (speedup 1.0000x reference)
import math

import jax
import jax.numpy as jnp
from jax.experimental import pallas as pl
from jax.experimental.pallas import tpu as pltpu

N = 4096
H = 256
W = 256

_TS = 16
_GT = 16
_NBR = 7
_C = 128
_NP = 256
_PADN = N + _GT * _GT * 7 + _C


def _raster_tile(starts_ref, ends_ref, params_ref, out_ref):
    ti = pl.program_id(0)
    tj = pl.program_id(1)
    t = ti * _GT + tj
    cx0 = (tj * _TS).astype(jnp.float32) + 8.0
    cy0 = (ti * _TS).astype(jnp.float32) + 8.0

    q = jax.lax.broadcasted_iota(jnp.int32, (_NP, 1), 0)
    xl = (q & (_TS - 1)).astype(jnp.float32) - 7.5
    yl = (q >> 4).astype(jnp.float32) - 7.5
    Q = jnp.concatenate(
        [xl * xl, xl, yl * yl, yl, xl * yl, jnp.ones_like(xl)],
        axis=1)
    lane_base = jax.lax.broadcasted_iota(jnp.int32, (1, _C), 1)

    def seg_contrib(acc, s, e):
        nch = (e - s + _C - 1) // _C

        def body(ci, acc):
            off = pl.multiple_of(s + ci * _C, 8)
            blk = params_ref[pl.ds(off, _C), :].T
            xr = blk[0:1, :] - cx0
            yr = blk[1:2, :] - cy0
            cA = blk[2:3, :]
            cB = blk[3:4, :]
            cC = blk[4:5, :]
            G = jnp.concatenate(
                [0.5 * cA,
                 -(cA * xr + cB * yr),
                 0.5 * cC,
                 -(cC * yr + cB * xr),
                 cB,
                 0.5 * cA * xr * xr + 0.5 * cC * yr * yr + cB * xr * yr],
                axis=0)
            sigma = jax.lax.dot_general(
                Q, G, (((1,), (0,)), ((), ())),
                preferred_element_type=jnp.float32)
            vals = jnp.where(sigma >= 0.0, jnp.exp(-sigma), 0.0)
            feat = jnp.where(lane_base + off < e, blk[5:8, :], 0.0)
            return acc + jax.lax.dot_general(
                feat, vals, (((1,), (1,)), ((), ())),
                preferred_element_type=jnp.float32)

        return jax.lax.fori_loop(0, nch, body, acc)

    acc = jnp.zeros((3, _NP), jnp.float32)
    for k in range(_NBR):
        acc = seg_contrib(acc, starts_ref[t, k], ends_ref[t, k])
    out_ref[...] = jnp.clip(acc, 0.0, 1.0).reshape(1, 1, 3, _TS, _TS)


def kernel(xyz, scaling, rotation, features, opacity):
    xc = 0.5 * (xyz[:, 0] + 1.0) * W
    yc = 0.5 * (xyz[:, 1] + 1.0) * H
    s = jnp.abs(scaling)
    theta = jax.nn.sigmoid(rotation[:, 0]) * (2.0 * math.pi)
    c = jnp.cos(theta)
    sn = jnp.sin(theta)
    sx2 = s[:, 0] ** 2
    sy2 = s[:, 1] ** 2
    Sxx = c * c * sx2 + sn * sn * sy2
    Sxy = c * sn * (sx2 - sy2)
    Syy = sn * sn * sx2 + c * c * sy2
    det = Sxx * Syy - Sxy * Sxy
    inv = 1.0 / (det + 1e-12)
    cA = Syy * inv
    cB = -Sxy * inv
    cC = Sxx * inv
    wf = features * opacity

    bx = jnp.clip((xc * (1.0 / _TS)).astype(jnp.int32), 0, _GT - 1)
    by = jnp.clip((yc * (1.0 / _TS)).astype(jnp.int32), 0, _GT - 1)
    bins = by * _GT + bx
    order = jnp.argsort(bins)
    sorted_bins = bins[order]
    starts = jnp.searchsorted(
        sorted_bins, jnp.arange(_GT * _GT + 1, dtype=jnp.int32)
    ).astype(jnp.int32)
    counts = starts[1:] - starts[:-1]
    pcounts = ((counts + 7) // 8) * 8
    pstarts = jnp.concatenate(
        [jnp.zeros((1,), jnp.int32), jnp.cumsum(pcounts, dtype=jnp.int32)])

    params = jnp.stack(
        [xc, yc, cA, cB, cC, wf[:, 0], wf[:, 1], wf[:, 2]], axis=1)
    pos = pstarts[sorted_bins] + (
        jnp.arange(N, dtype=jnp.int32) - starts[sorted_bins])
    padded = jnp.zeros((_PADN, 8), jnp.float32).at[pos].set(params[order])

    ti = jnp.arange(_GT, dtype=jnp.int32)
    rows = ti[:, None] + jnp.arange(-3, 4, dtype=jnp.int32)[None, :]
    valid = (rows >= 0) & (rows < _GT)
    rows_c = jnp.clip(rows, 0, _GT - 1)
    c0 = jnp.clip(ti - 3, 0, _GT - 1)
    c1 = jnp.clip(ti + 3, 0, _GT - 1)
    seg_s = pstarts[rows_c[:, None, :] * _GT + c0[None, :, None]]
    seg_e = pstarts[rows_c[:, None, :] * _GT + c1[None, :, None] + 1]
    valid3 = jnp.broadcast_to(valid[:, None, :], (_GT, _GT, _NBR))
    seg_s = jnp.where(valid3, seg_s, 0).reshape(_GT * _GT, _NBR)
    seg_e = jnp.where(valid3, seg_e, 0).reshape(_GT * _GT, _NBR)

    grid_spec = pltpu.PrefetchScalarGridSpec(
        num_scalar_prefetch=2,
        grid=(_GT, _GT),
        in_specs=[pl.BlockSpec((_PADN, 8), lambda i, j, *_: (0, 0))],
        out_specs=pl.BlockSpec((1, 1, 3, _TS, _TS),
                               lambda i, j, *_: (i, j, 0, 0, 0)),
    )
    img = pl.pallas_call(
        _raster_tile,
        grid_spec=grid_spec,
        out_shape=jax.ShapeDtypeStruct((_GT, _GT, 3, _TS, _TS), jnp.float32),
    )(seg_s, seg_e, padded)
    return img.transpose(2, 0, 3, 1, 4).reshape(1, 3, H, W)


# speedup vs baseline: 2.0593x; 2.0593x over previous
import math

import jax
import jax.numpy as jnp
from jax.experimental import pallas as pl
from jax.experimental.pallas import tpu as pltpu

N = 4096
H = 256
W = 256

_TS = 16
_GT = 16
_C = 128
_NP = 256
_NPAIR = N * 7
_NBLK = _NPAIR // _C + 2


def _raster_tile(starts_ref, ends_ref, params_ref, out_ref):
    ti = pl.program_id(0)
    tj = pl.program_id(1)
    t = ti * _GT + tj
    cx0 = (tj * _TS).astype(jnp.float32) + 8.0
    cy0 = (ti * _TS).astype(jnp.float32) + 8.0

    q = jax.lax.broadcasted_iota(jnp.int32, (_NP, 1), 0)
    xl = (q & (_TS - 1)).astype(jnp.float32) - 7.5
    yl = (q >> 4).astype(jnp.float32) - 7.5
    Q = jnp.concatenate(
        [xl * xl, xl, yl * yl, yl, xl * yl, jnp.ones_like(xl)],
        axis=1)
    lane = jax.lax.broadcasted_iota(jnp.int32, (1, _C), 1)

    s = starts_ref[t]
    e = ends_ref[t]
    b0 = s // _C
    nit = (e - b0 * _C + 2 * _C - 1) // (2 * _C)

    def one_block(b):
        blk = params_ref[b, :, :]
        xr = blk[0:1, :] - cx0
        yr = blk[1:2, :] - cy0
        cA = blk[2:3, :]
        cB = blk[3:4, :]
        cC = blk[4:5, :]
        G = jnp.concatenate(
            [0.5 * cA,
             -(cA * xr + cB * yr),
             0.5 * cC,
             -(cC * yr + cB * xr),
             cB,
             0.5 * cA * xr * xr + 0.5 * cC * yr * yr + cB * xr * yr],
            axis=0)
        sigma = jax.lax.dot_general(
            Q, G, (((1,), (0,)), ((), ())),
            preferred_element_type=jnp.float32)
        vals = jnp.where(sigma >= 0.0, jnp.exp(-sigma), 0.0)
        colid = lane + b * _C
        feat = jnp.where((colid >= s) & (colid < e), blk[5:8, :], 0.0)
        return jax.lax.dot_general(
            feat, vals, (((1,), (1,)), ((), ())),
            preferred_element_type=jnp.float32)

    def body(ci, acc):
        b = b0 + 2 * ci
        return acc + one_block(b) + one_block(b + 1)

    acc = jax.lax.fori_loop(0, nit, body, jnp.zeros((3, _NP), jnp.float32))
    out_ref[...] = jnp.clip(acc, 0.0, 1.0).reshape(1, 1, 3, _TS, _TS)


def kernel(xyz, scaling, rotation, features, opacity):
    xc = 0.5 * (xyz[:, 0] + 1.0) * W
    yc = 0.5 * (xyz[:, 1] + 1.0) * H
    s = jnp.abs(scaling)
    theta = jax.nn.sigmoid(rotation[:, 0]) * (2.0 * math.pi)
    c = jnp.cos(theta)
    sn = jnp.sin(theta)
    sx2 = s[:, 0] ** 2
    sy2 = s[:, 1] ** 2
    Sxx = c * c * sx2 + sn * sn * sy2
    Sxy = c * sn * (sx2 - sy2)
    Syy = sn * sn * sx2 + c * c * sy2
    det = Sxx * Syy - Sxy * Sxy
    inv = 1.0 / (det + 1e-12)
    cA = Syy * inv
    cB = -Sxy * inv
    cC = Sxx * inv
    wf = features * opacity

    bx = jnp.clip((xc * (1.0 / _TS)).astype(jnp.int32), 0, _GT - 1)
    by = jnp.clip((yc * (1.0 / _TS)).astype(jnp.int32), 0, _GT - 1)
    gid = jnp.arange(N, dtype=jnp.int32)
    band = by[:, None] + jnp.arange(-3, 4, dtype=jnp.int32)[None, :]
    group = band * _GT + bx[:, None]
    key = jnp.where((band >= 0) & (band < _GT),
                    group * 4096 + gid[:, None],
                    (1 << 29) + gid[:, None])
    skeys = jnp.sort(key.ravel())
    gidx = skeys & 4095
    sgroup = skeys >> 12

    params = jnp.stack(
        [xc, yc, cA, cB, cC, wf[:, 0], wf[:, 1], wf[:, 2]], axis=1)
    pairs = params[gidx]
    pairs = jnp.pad(pairs, ((0, _NBLK * _C - _NPAIR), (0, 0)))
    pairs = pairs.reshape(_NBLK, _C, 8).transpose(0, 2, 1)

    gstarts = jnp.searchsorted(
        sgroup, jnp.arange(_GT * _GT + 1, dtype=jnp.int32)
    ).astype(jnp.int32)

    ti = jnp.arange(_GT, dtype=jnp.int32)
    c0 = jnp.clip(ti - 3, 0, _GT - 1)
    c1 = jnp.clip(ti + 3, 0, _GT - 1)
    seg_s = gstarts[ti[:, None] * _GT + c0[None, :]].reshape(-1)
    seg_e = gstarts[ti[:, None] * _GT + c1[None, :] + 1].reshape(-1)

    grid_spec = pltpu.PrefetchScalarGridSpec(
        num_scalar_prefetch=2,
        grid=(_GT, _GT),
        in_specs=[pl.BlockSpec((_NBLK, 8, _C), lambda i, j, *_: (0, 0, 0))],
        out_specs=pl.BlockSpec((1, 1, 3, _TS, _TS),
                               lambda i, j, *_: (i, j, 0, 0, 0)),
    )
    img = pl.pallas_call(
        _raster_tile,
        grid_spec=grid_spec,
        out_shape=jax.ShapeDtypeStruct((_GT, _GT, 3, _TS, _TS), jnp.float32),
    )(seg_s, seg_e, pairs)
    return img.transpose(2, 0, 3, 1, 4).reshape(1, 3, H, W)


# speedup vs baseline: 2.1163x; 1.0277x over previous
import math

import jax
import jax.numpy as jnp
from jax.experimental import pallas as pl
from jax.experimental.pallas import tpu as pltpu

N = 4096
H = 256
W = 256

_TS = 16
_GT = 16
_C = 128
_NP = 256
_NPAIR = N * 7
_NBLK = _NPAIR // _C + 2


def _raster_tile(starts_ref, ends_ref, q_ref, params_ref, out_ref):
    ti = pl.program_id(0)
    tj = pl.program_id(1)
    t = ti * _GT + tj
    cx0 = (tj * _TS).astype(jnp.float32) + 8.0
    cy0 = (ti * _TS).astype(jnp.float32) + 8.0

    Q = q_ref[...]
    lane = jax.lax.broadcasted_iota(jnp.int32, (1, _C), 1)

    s = starts_ref[t]
    e = ends_ref[t]
    b0 = s // _C
    nit = (e - b0 * _C + 2 * _C - 1) // (2 * _C)

    def one_block(b):
        blk = params_ref[b, :, :]
        xr = blk[0:1, :] - cx0
        yr = blk[1:2, :] - cy0
        cA = blk[2:3, :]
        cB = blk[3:4, :]
        cC = blk[4:5, :]
        G = jnp.concatenate(
            [0.5 * cA,
             -(cA * xr + cB * yr),
             0.5 * cC,
             -(cC * yr + cB * xr),
             cB,
             0.5 * cA * xr * xr + 0.5 * cC * yr * yr + cB * xr * yr],
            axis=0)
        sigma = jax.lax.dot_general(
            Q, G, (((1,), (0,)), ((), ())),
            preferred_element_type=jnp.float32)
        vals = jnp.where(sigma >= 0.0, jnp.exp(-sigma), 0.0)
        colid = lane + b * _C
        feat = jnp.where((colid >= s) & (colid < e), blk[5:8, :], 0.0)
        return jax.lax.dot_general(
            feat, vals, (((1,), (1,)), ((), ())),
            preferred_element_type=jnp.float32)

    def body(ci, accs):
        a1, a2 = accs
        b = b0 + 2 * ci
        return (a1 + one_block(b), a2 + one_block(b + 1))

    z = jnp.zeros((3, _NP), jnp.float32)
    a1, a2 = jax.lax.fori_loop(0, nit, body, (z, z))
    out_ref[...] = jnp.clip(a1 + a2, 0.0, 1.0).reshape(1, 1, 3, _TS, _TS)


def kernel(xyz, scaling, rotation, features, opacity):
    xc = 0.5 * (xyz[:, 0] + 1.0) * W
    yc = 0.5 * (xyz[:, 1] + 1.0) * H
    s = jnp.abs(scaling)
    theta = jax.nn.sigmoid(rotation[:, 0]) * (2.0 * math.pi)
    c = jnp.cos(theta)
    sn = jnp.sin(theta)
    sx2 = s[:, 0] ** 2
    sy2 = s[:, 1] ** 2
    Sxx = c * c * sx2 + sn * sn * sy2
    Sxy = c * sn * (sx2 - sy2)
    Syy = sn * sn * sx2 + c * c * sy2
    det = Sxx * Syy - Sxy * Sxy
    inv = 1.0 / (det + 1e-12)
    cA = Syy * inv
    cB = -Sxy * inv
    cC = Sxx * inv
    wf = features * opacity

    bx = jnp.clip((xc * (1.0 / _TS)).astype(jnp.int32), 0, _GT - 1)
    by = jnp.clip((yc * (1.0 / _TS)).astype(jnp.int32), 0, _GT - 1)
    gid = jnp.arange(N, dtype=jnp.int32)
    band = by[:, None] + jnp.arange(-3, 4, dtype=jnp.int32)[None, :]
    group = band * _GT + bx[:, None]
    key = jnp.where((band >= 0) & (band < _GT),
                    group * 4096 + gid[:, None],
                    (1 << 29) + gid[:, None])
    skeys = jnp.sort(key.ravel())
    gidx = skeys & 4095
    sgroup = skeys >> 12

    params = jnp.stack(
        [xc, yc, cA, cB, cC, wf[:, 0], wf[:, 1], wf[:, 2]], axis=1)
    pairs = params[gidx]
    pairs = jnp.pad(pairs, ((0, _NBLK * _C - _NPAIR), (0, 0)))
    pairs = pairs.reshape(_NBLK, _C, 8).transpose(0, 2, 1)

    gstarts = jnp.searchsorted(
        sgroup, jnp.arange(_GT * _GT + 1, dtype=jnp.int32)
    ).astype(jnp.int32)

    ti = jnp.arange(_GT, dtype=jnp.int32)
    c0 = jnp.clip(ti - 3, 0, _GT - 1)
    c1 = jnp.clip(ti + 3, 0, _GT - 1)
    seg_s = gstarts[ti[:, None] * _GT + c0[None, :]].reshape(-1)
    seg_e = gstarts[ti[:, None] * _GT + c1[None, :] + 1].reshape(-1)

    p = jnp.arange(_NP, dtype=jnp.int32)
    xl = (p & (_TS - 1)).astype(jnp.float32) - 7.5
    yl = (p >> 4).astype(jnp.float32) - 7.5
    qmat = jnp.stack(
        [xl * xl, xl, yl * yl, yl, xl * yl, jnp.ones_like(xl)],
        axis=1)

    grid_spec = pltpu.PrefetchScalarGridSpec(
        num_scalar_prefetch=2,
        grid=(_GT, _GT),
        in_specs=[
            pl.BlockSpec((_NP, 6), lambda i, j, *_: (0, 0)),
            pl.BlockSpec((_NBLK, 8, _C), lambda i, j, *_: (0, 0, 0)),
        ],
        out_specs=pl.BlockSpec((1, 1, 3, _TS, _TS),
                               lambda i, j, *_: (i, j, 0, 0, 0)),
    )
    img = pl.pallas_call(
        _raster_tile,
        grid_spec=grid_spec,
        out_shape=jax.ShapeDtypeStruct((_GT, _GT, 3, _TS, _TS), jnp.float32),
    )(seg_s, seg_e, qmat, pairs)
    return img.transpose(2, 0, 3, 1, 4).reshape(1, 3, H, W)


# speedup vs baseline: 2.4257x; 1.1462x over previous
import math

import jax
import jax.numpy as jnp
from jax.experimental import pallas as pl
from jax.experimental.pallas import tpu as pltpu

N = 4096
H = 256
W = 256

_TS = 16
_GT = 16
_C = 128
_NP = 256
_NPAIR = N * 7
_NBLK = _NPAIR // _C + 2


def _raster_tile(starts_ref, ends_ref, q_ref, params_ref, out_ref):
    ti = pl.program_id(0)
    tj = pl.program_id(1)
    t = ti * _GT + tj
    cx0 = (tj * _TS).astype(jnp.float32) + 8.0
    cy0 = (ti * _TS).astype(jnp.float32) + 8.0

    Q = q_ref[...]
    lane = jax.lax.broadcasted_iota(jnp.int32, (1, _C), 1)

    s = starts_ref[t]
    e = ends_ref[t]
    b0 = s // _C
    nit = (e - b0 * _C + 2 * _C - 1) // (2 * _C)

    def one_block(b):
        blk = params_ref[b, :, :]
        xr = blk[0:1, :] - cx0
        yr = blk[1:2, :] - cy0
        cA = blk[2:3, :]
        cB = blk[3:4, :]
        cC = blk[4:5, :]
        G = jnp.concatenate(
            [0.5 * cA,
             -(cA * xr + cB * yr),
             0.5 * cC,
             -(cC * yr + cB * xr),
             cB,
             0.5 * cA * xr * xr + 0.5 * cC * yr * yr + cB * xr * yr],
            axis=0)
        sigma = jax.lax.dot_general(
            Q, G, (((1,), (0,)), ((), ())),
            preferred_element_type=jnp.float32)
        vals = jnp.where(sigma >= 0.0, jnp.exp(-sigma), 0.0)
        colid = lane + b * _C
        feat = jnp.where((colid >= s) & (colid < e), blk[5:8, :], 0.0)
        return jax.lax.dot_general(
            feat, vals, (((1,), (1,)), ((), ())),
            preferred_element_type=jnp.float32)

    def body(ci, accs):
        a1, a2 = accs
        b = b0 + 2 * ci
        return (a1 + one_block(b), a2 + one_block(b + 1))

    z = jnp.zeros((3, _NP), jnp.float32)
    a1, a2 = jax.lax.fori_loop(0, nit, body, (z, z))
    out_ref[...] = jnp.clip(a1 + a2, 0.0, 1.0).reshape(1, 1, 3, _TS, _TS)


def kernel(xyz, scaling, rotation, features, opacity):
    xc = 0.5 * (xyz[:, 0] + 1.0) * W
    yc = 0.5 * (xyz[:, 1] + 1.0) * H
    s = jnp.abs(scaling)
    theta = jax.nn.sigmoid(rotation[:, 0]) * (2.0 * math.pi)
    c = jnp.cos(theta)
    sn = jnp.sin(theta)
    sx2 = s[:, 0] ** 2
    sy2 = s[:, 1] ** 2
    Sxx = c * c * sx2 + sn * sn * sy2
    Sxy = c * sn * (sx2 - sy2)
    Syy = sn * sn * sx2 + c * c * sy2
    det = Sxx * Syy - Sxy * Sxy
    inv = 1.0 / (det + 1e-12)
    cA = Syy * inv
    cB = -Sxy * inv
    cC = Sxx * inv
    wf = features * opacity

    bx = jnp.clip((xc * (1.0 / _TS)).astype(jnp.int32), 0, _GT - 1)
    by = jnp.clip((yc * (1.0 / _TS)).astype(jnp.int32), 0, _GT - 1)
    gid = jnp.arange(N, dtype=jnp.int32)
    dd = jnp.arange(-3, 4, dtype=jnp.int32)
    band = by[:, None] + dd[None, :]
    group = band * _GT + bx[:, None]
    ry = jnp.sqrt(24.0 * Syy)
    thr = ((jnp.abs(dd) - 1) * _TS).astype(jnp.float32)
    keep = (band >= 0) & (band < _GT) & (ry[:, None] > thr[None, :])
    key = jnp.where(keep,
                    group * 4096 + gid[:, None],
                    (1 << 29) + gid[:, None])
    skeys = jnp.sort(key.ravel())
    gidx = skeys & 4095
    sgroup = skeys >> 12

    params = jnp.stack(
        [xc, yc, cA, cB, cC, wf[:, 0], wf[:, 1], wf[:, 2]], axis=1)
    pairs = params[gidx]
    pairs = jnp.pad(pairs, ((0, _NBLK * _C - _NPAIR), (0, 0)))
    pairs = pairs.reshape(_NBLK, _C, 8).transpose(0, 2, 1)

    gstarts = jnp.searchsorted(
        sgroup, jnp.arange(_GT * _GT + 1, dtype=jnp.int32)
    ).astype(jnp.int32)

    ti = jnp.arange(_GT, dtype=jnp.int32)
    c0 = jnp.clip(ti - 3, 0, _GT - 1)
    c1 = jnp.clip(ti + 3, 0, _GT - 1)
    seg_s = gstarts[ti[:, None] * _GT + c0[None, :]].reshape(-1)
    seg_e = gstarts[ti[:, None] * _GT + c1[None, :] + 1].reshape(-1)

    p = jnp.arange(_NP, dtype=jnp.int32)
    xl = (p & (_TS - 1)).astype(jnp.float32) - 7.5
    yl = (p >> 4).astype(jnp.float32) - 7.5
    qmat = jnp.stack(
        [xl * xl, xl, yl * yl, yl, xl * yl, jnp.ones_like(xl)],
        axis=1)

    grid_spec = pltpu.PrefetchScalarGridSpec(
        num_scalar_prefetch=2,
        grid=(_GT, _GT),
        in_specs=[
            pl.BlockSpec((_NP, 6), lambda i, j, *_: (0, 0)),
            pl.BlockSpec((_NBLK, 8, _C), lambda i, j, *_: (0, 0, 0)),
        ],
        out_specs=pl.BlockSpec((1, 1, 3, _TS, _TS),
                               lambda i, j, *_: (i, j, 0, 0, 0)),
    )
    img = pl.pallas_call(
        _raster_tile,
        grid_spec=grid_spec,
        out_shape=jax.ShapeDtypeStruct((_GT, _GT, 3, _TS, _TS), jnp.float32),
    )(seg_s, seg_e, qmat, pairs)
    return img.transpose(2, 0, 3, 1, 4).reshape(1, 3, H, W)
